# baseline (speedup 1.0000x reference)
import functools

import jax
import jax.numpy as jnp
from jax import lax
from jax.experimental import pallas as pl
from jax.experimental.pallas import tpu as pltpu
from jax.experimental.pallas import tpu_sc as plsc

_T = 2048
_D = 1024
_F = 1024
_E = 8
_P = 2 * _T
_BLK = 128
_NBMAX = (_P // _BLK) + _E - 1
_CAP = _NBMAX * _BLK
_CHUNK = 256
_NCHUNK = _P // _CHUNK
_LOAD_COEF = 0.01
_EPS = 1e-6


def _router_kernel(flat_ref, u_ref, wr_ref, sb_ref, gb_ref,
                   pv_ref, slot_ref, meta_ref, bal_ref):
    flat = flat_ref[...]
    logits = lax.dot_general(flat, wr_ref[...], (((1,), (1,)), ((), ())),
                             preferred_element_type=jnp.float32)
    u = jnp.clip(u_ref[...], 0.0, 1.0)
    logits = logits + u * sb_ref[...] + (1.0 - u) * gb_ref[...]
    m = jnp.max(logits, axis=-1, keepdims=True)
    ex = jnp.exp(logits - m)
    probs = ex / jnp.sum(ex, axis=-1, keepdims=True)

    lane = lax.broadcasted_iota(jnp.int32, probs.shape, 1)
    m1 = jnp.max(probs, axis=-1, keepdims=True)
    i1 = jnp.min(jnp.where(probs == m1, lane, _E), axis=-1, keepdims=True)
    oh1 = (lane == i1)
    p_rest = jnp.where(oh1, -jnp.inf, probs)
    m2 = jnp.max(p_rest, axis=-1, keepdims=True)
    i2 = jnp.min(jnp.where(p_rest == m2, lane, _E), axis=-1, keepdims=True)
    oh2 = (lane == i2)

    pv_ref[0:_T, :] = m1
    pv_ref[_T:_P, :] = m2

    colmean = jnp.mean(probs, axis=0, keepdims=True)
    bal_ref[...] = _E * _LOAD_COEF * jnp.sum(colmean * colmean, keepdims=True)

    oh = jnp.concatenate([jnp.where(oh1, 1.0, 0.0),
                          jnp.where(oh2, 1.0, 0.0)], axis=0)

    r = lax.broadcasted_iota(jnp.int32, (_CHUNK, _CHUNK), 0)
    c = lax.broadcasted_iota(jnp.int32, (_CHUNK, _CHUNK), 1)
    ltri = jnp.where(r >= c, 1.0, 0.0)

    incs = []
    tails = []
    for ci in range(_NCHUNK):
        chunk = oh[ci * _CHUNK:(ci + 1) * _CHUNK, :]
        inc = jnp.dot(ltri, chunk, preferred_element_type=jnp.float32)
        incs.append(inc)
        tails.append(inc[_CHUNK - 1:_CHUNK, :])
    totals = jnp.concatenate(tails, axis=0)

    r2 = lax.broadcasted_iota(jnp.int32, (_NCHUNK, _NCHUNK), 0)
    c2 = lax.broadcasted_iota(jnp.int32, (_NCHUNK, _NCHUNK), 1)
    sltri = jnp.where(r2 > c2, 1.0, 0.0)
    exc_chunk = jnp.dot(sltri, totals, preferred_element_type=jnp.float32)

    counts = jnp.sum(totals, axis=0, keepdims=True)
    nb = jnp.floor((counts + float(_BLK - 1)) * (1.0 / _BLK))
    eye_c = lax.broadcasted_iota(jnp.int32, (_E, _E), 0)
    eye_r = lax.broadcasted_iota(jnp.int32, (_E, _E), 1)
    supper = jnp.where(eye_c < eye_r, 1.0, 0.0)
    blk_start = jnp.dot(nb, supper, preferred_element_type=jnp.float32)
    used = jnp.sum(nb, axis=1, keepdims=True)

    base_lane = blk_start * float(_BLK)
    slot_f = []
    for ci in range(_NCHUNK):
        ohc = oh[ci * _CHUNK:(ci + 1) * _CHUNK, :]
        cum_exc = incs[ci] - ohc + exc_chunk[ci:ci + 1, :]
        slot_f.append(jnp.sum(ohc * (cum_exc + base_lane), axis=1,
                              keepdims=True))
    slot = jnp.concatenate(slot_f, axis=0)
    slot_ref[...] = slot.astype(jnp.int32)

    bi = lax.broadcasted_iota(jnp.int32, (_NBMAX, _E), 0).astype(
        jnp.float32)
    bi_cl = jnp.minimum(bi, used[0, 0] - 1.0)
    be = jnp.sum(jnp.where(blk_start <= bi_cl, 1.0, 0.0), axis=1,
                 keepdims=True) - 1.0
    xi = bi_cl[:, 0:1]
    meta_ref[0:1, :] = used.astype(jnp.int32)
    meta_ref[1:1 + _NBMAX, :] = be.astype(jnp.int32)
    meta_ref[1 + _NBMAX:1 + 2 * _NBMAX, :] = xi.astype(jnp.int32)


def _swiglu(xb, wg, wu, wd):
    g = lax.dot_general(xb, wg, (((1,), (1,)), ((), ())),
                        preferred_element_type=jnp.float32)
    u = lax.dot_general(xb, wu, (((1,), (1,)), ((), ())),
                        preferred_element_type=jnp.float32)
    h = ((g * lax.logistic(g)) * u).astype(jnp.bfloat16)
    return lax.dot_general(h, wd, (((1,), (1,)), ((), ())),
                           preferred_element_type=jnp.float32)


def _grouped_ffn_kernel(meta_ref, x_ref, wg_ref, wu_ref, wd_ref, out_ref):
    i = pl.program_id(0)

    @pl.when(i < meta_ref[0])
    def _():
        out_ref[...] = _swiglu(x_ref[...].astype(jnp.bfloat16),
                               wg_ref[0].astype(jnp.bfloat16),
                               wu_ref[0].astype(jnp.bfloat16),
                               wd_ref[0].astype(jnp.bfloat16))


def _shared_kernel(x_ref, sg_ref, su_ref, sd_ref, out_ref):
    xb = x_ref[...].astype(jnp.bfloat16)
    out_ref[...] = _swiglu(xb, sg_ref[0].astype(jnp.bfloat16),
                           su_ref[0].astype(jnp.bfloat16),
                           sd_ref[0].astype(jnp.bfloat16))


def _combine_kernel(pv_ref1, pv_ref2, y1_ref, y2_ref, sh_ref, nw_ref,
                    out_ref):
    y = (pv_ref1[...] * y1_ref[...] + pv_ref2[...] * y2_ref[...]
         + sh_ref[...])
    rms = lax.rsqrt(jnp.mean(y * y, axis=-1, keepdims=True) + _EPS)
    out_ref[...] = y * rms * nw_ref[...]


_ROWS_PER_TILE = _P // 32
_SUB = 32
_NSUB = _ROWS_PER_TILE // _SUB


@functools.lru_cache(maxsize=None)
def _get_sc_kernels():
    mesh = plsc.VectorSubcoreMesh(core_axis_name="c", subcore_axis_name="s")

    @functools.partial(
        pl.kernel, mesh=mesh,
        out_type=jax.ShapeDtypeStruct((_CAP, _D), jnp.float32),
        scratch_types=[pltpu.VMEM((_NSUB, _SUB), jnp.int32),
                       pltpu.VMEM((_SUB, _D), jnp.float32),
                       pltpu.VMEM((_SUB, _D), jnp.float32),
                       pltpu.SemaphoreType.DMA,
                       pltpu.SemaphoreType.DMA,
                       pltpu.SemaphoreType.DMA,
                       pltpu.SemaphoreType.DMA],
    )
    def dispatch_sc(flat_hbm, slot_hbm, xs_hbm,
                    idx_v, rows0, rows1, sl0, sl1, ss0, ss1):
        wid = lax.axis_index("s") * 2 + lax.axis_index("c")
        tok0 = (wid % 16) * _ROWS_PER_TILE
        rows = (rows0, rows1)
        lsem = (sl0, sl1)
        ssem = (ss0, ss1)
        pltpu.sync_copy(slot_hbm.at[wid], idx_v)
        loads = [None, None]
        scat = [None, None]
        loads[0] = pltpu.async_copy(
            flat_hbm.at[pl.ds(tok0, _SUB)], rows0, lsem[0])
        for cc in range(_NSUB):
            b = cc % 2
            nb_ = (cc + 1) % 2
            loads[b].wait()
            if cc + 1 < _NSUB:
                if scat[nb_] is not None:
                    scat[nb_].wait()
                loads[nb_] = pltpu.async_copy(
                    flat_hbm.at[pl.ds(tok0 + (cc + 1) * _SUB, _SUB)],
                    rows[nb_], lsem[nb_])
            scat[b] = pltpu.async_copy(rows[b], xs_hbm.at[idx_v.at[cc]],
                                       ssem[b])
        for b in range(2):
            if scat[b] is not None:
                scat[b].wait()

    @functools.partial(
        pl.kernel, mesh=mesh,
        out_type=jax.ShapeDtypeStruct((_P, _D), jnp.float32),
        scratch_types=[pltpu.VMEM((_NSUB, _SUB), jnp.int32),
                       pltpu.VMEM((_SUB, _D), jnp.float32),
                       pltpu.VMEM((_SUB, _D), jnp.float32),
                       pltpu.SemaphoreType.DMA,
                       pltpu.SemaphoreType.DMA,
                       pltpu.SemaphoreType.DMA,
                       pltpu.SemaphoreType.DMA],
    )
    def combine_gather_sc(eo_hbm, slot_hbm, y12_hbm,
                          idx_v, rows0, rows1, sg0, sg1, sw0, sw1):
        wid = lax.axis_index("s") * 2 + lax.axis_index("c")
        base = wid * _ROWS_PER_TILE
        rows = (rows0, rows1)
        gsem = (sg0, sg1)
        wsem = (sw0, sw1)
        pltpu.sync_copy(slot_hbm.at[wid], idx_v)
        gath = [None, None]
        wr = [None, None]
        gath[0] = pltpu.async_copy(eo_hbm.at[idx_v.at[0]], rows0, gsem[0])
        for cc in range(_NSUB):
            b = cc % 2
            nb_ = (cc + 1) % 2
            gath[b].wait()
            if cc + 1 < _NSUB:
                if wr[nb_] is not None:
                    wr[nb_].wait()
                gath[nb_] = pltpu.async_copy(
                    eo_hbm.at[idx_v.at[cc + 1]], rows[nb_], gsem[nb_])
            wr[b] = pltpu.async_copy(
                rows[b], y12_hbm.at[pl.ds(base + cc * _SUB, _SUB)], wsem[b])
        for b in range(2):
            if wr[b] is not None:
                wr[b].wait()

    return dispatch_sc, combine_gather_sc


@jax.jit
def kernel(x, U, Wr, spec_bias, gen_bias, Wg, Wu, Wd, Sg, Su, Sd, norm_w):
    flat = x.reshape(_T, _D)
    u_col = U.reshape(_T, 1)

    pv, slot, meta, bal = pl.pallas_call(
        _router_kernel,
        out_shape=(
            jax.ShapeDtypeStruct((_P, 1), jnp.float32),
            jax.ShapeDtypeStruct((_P, 1), jnp.int32),
            jax.ShapeDtypeStruct((1 + 2 * _NBMAX, 1), jnp.int32),
            jax.ShapeDtypeStruct((1, 1), jnp.float32),
        ),
    )(flat, u_col, Wr, spec_bias.reshape(1, _E), gen_bias.reshape(1, _E))

    slot_tiled = slot.reshape(32, _NSUB, _SUB)
    meta_flat = meta.reshape(1 + 2 * _NBMAX)
    dispatch_sc, combine_gather_sc = _get_sc_kernels()

    x_sorted = dispatch_sc(flat, slot_tiled)

    shared_out = pl.pallas_call(
        _shared_kernel,
        grid=(2,),
        in_specs=[
            pl.BlockSpec((_T // 2, _D), lambda t: (t, 0)),
            pl.BlockSpec((1, _F, _D), lambda t: (0, 0, 0)),
            pl.BlockSpec((1, _F, _D), lambda t: (0, 0, 0)),
            pl.BlockSpec((1, _D, _F), lambda t: (0, 0, 0)),
        ],
        out_specs=pl.BlockSpec((_T // 2, _D), lambda t: (t, 0)),
        out_shape=jax.ShapeDtypeStruct((_T, _D), jnp.float32),
    )(flat, Sg, Su, Sd)

    nb1 = _NBMAX + 1
    eo_sorted = pl.pallas_call(
        _grouped_ffn_kernel,
        grid_spec=pltpu.PrefetchScalarGridSpec(
            num_scalar_prefetch=1,
            grid=(_NBMAX,),
            in_specs=[
                pl.BlockSpec((_BLK, _D), lambda i, m: (m[nb1 + i], 0)),
                pl.BlockSpec((1, _F, _D), lambda i, m: (m[1 + i], 0, 0)),
                pl.BlockSpec((1, _F, _D), lambda i, m: (m[1 + i], 0, 0)),
                pl.BlockSpec((1, _D, _F), lambda i, m: (m[1 + i], 0, 0)),
            ],
            out_specs=pl.BlockSpec((_BLK, _D), lambda i, m: (m[nb1 + i], 0)),
        ),
        out_shape=jax.ShapeDtypeStruct((_CAP, _D), jnp.float32),
    )(meta_flat, x_sorted, Wg, Wu, Wd)

    y12 = combine_gather_sc(eo_sorted, slot_tiled)

    y = pl.pallas_call(
        _combine_kernel,
        grid=(2,),
        in_specs=[
            pl.BlockSpec((_T // 2, 1), lambda t: (t, 0)),
            pl.BlockSpec((_T // 2, 1), lambda t: (t + 2, 0)),
            pl.BlockSpec((_T // 2, _D), lambda t: (t, 0)),
            pl.BlockSpec((_T // 2, _D), lambda t: (t + 2, 0)),
            pl.BlockSpec((_T // 2, _D), lambda t: (t, 0)),
            pl.BlockSpec((1, _D), lambda t: (0, 0)),
        ],
        out_specs=pl.BlockSpec((_T // 2, _D), lambda t: (t, 0)),
        out_shape=jax.ShapeDtypeStruct((_T, _D), jnp.float32),
    )(pv, pv, y12, y12, shared_out, norm_w.reshape(1, _D))

    return (y.reshape(x.shape), bal.reshape(()))


# speedup vs baseline: 1.3367x; 1.3367x over previous
import functools

import jax
import jax.numpy as jnp
from jax import lax
from jax.experimental import pallas as pl
from jax.experimental.pallas import tpu as pltpu
from jax.experimental.pallas import tpu_sc as plsc

_T = 2048
_D = 1024
_F = 1024
_E = 8
_P = 2 * _T
_BLK = 512
_NBMAX = (_P // _BLK) + _E - 1
_CAP = _NBMAX * _BLK
_CHUNK = 256
_NCHUNK = _P // _CHUNK
_LOAD_COEF = 0.01
_EPS = 1e-6


def _router_kernel(flat_ref, u_ref, wr_ref, sb_ref, gb_ref,
                   pv_ref, slot_ref, meta_ref, bal_ref):
    flat = flat_ref[...]
    logits = lax.dot_general(flat, wr_ref[...], (((1,), (1,)), ((), ())),
                             preferred_element_type=jnp.float32)
    u = jnp.clip(u_ref[...], 0.0, 1.0)
    logits = logits + u * sb_ref[...] + (1.0 - u) * gb_ref[...]
    m = jnp.max(logits, axis=-1, keepdims=True)
    ex = jnp.exp(logits - m)
    probs = ex / jnp.sum(ex, axis=-1, keepdims=True)

    lane = lax.broadcasted_iota(jnp.int32, probs.shape, 1)
    m1 = jnp.max(probs, axis=-1, keepdims=True)
    i1 = jnp.min(jnp.where(probs == m1, lane, _E), axis=-1, keepdims=True)
    oh1 = (lane == i1)
    p_rest = jnp.where(oh1, -jnp.inf, probs)
    m2 = jnp.max(p_rest, axis=-1, keepdims=True)
    i2 = jnp.min(jnp.where(p_rest == m2, lane, _E), axis=-1, keepdims=True)
    oh2 = (lane == i2)

    pv_ref[0:_T, :] = m1
    pv_ref[_T:_P, :] = m2

    colmean = jnp.mean(probs, axis=0, keepdims=True)
    bal_ref[...] = _E * _LOAD_COEF * jnp.sum(colmean * colmean, keepdims=True)

    oh = jnp.concatenate([jnp.where(oh1, 1.0, 0.0),
                          jnp.where(oh2, 1.0, 0.0)], axis=0)

    r = lax.broadcasted_iota(jnp.int32, (_CHUNK, _CHUNK), 0)
    c = lax.broadcasted_iota(jnp.int32, (_CHUNK, _CHUNK), 1)
    ltri = jnp.where(r >= c, 1.0, 0.0)

    incs = []
    tails = []
    for ci in range(_NCHUNK):
        chunk = oh[ci * _CHUNK:(ci + 1) * _CHUNK, :]
        inc = jnp.dot(ltri, chunk, preferred_element_type=jnp.float32)
        incs.append(inc)
        tails.append(inc[_CHUNK - 1:_CHUNK, :])
    totals = jnp.concatenate(tails, axis=0)

    r2 = lax.broadcasted_iota(jnp.int32, (_NCHUNK, _NCHUNK), 0)
    c2 = lax.broadcasted_iota(jnp.int32, (_NCHUNK, _NCHUNK), 1)
    sltri = jnp.where(r2 > c2, 1.0, 0.0)
    exc_chunk = jnp.dot(sltri, totals, preferred_element_type=jnp.float32)

    counts = jnp.sum(totals, axis=0, keepdims=True)
    nb = jnp.floor((counts + float(_BLK - 1)) * (1.0 / _BLK))
    eye_c = lax.broadcasted_iota(jnp.int32, (_E, _E), 0)
    eye_r = lax.broadcasted_iota(jnp.int32, (_E, _E), 1)
    supper = jnp.where(eye_c < eye_r, 1.0, 0.0)
    blk_start = jnp.dot(nb, supper, preferred_element_type=jnp.float32)
    used = jnp.sum(nb, axis=1, keepdims=True)

    base_lane = blk_start * float(_BLK)
    slot_f = []
    for ci in range(_NCHUNK):
        ohc = oh[ci * _CHUNK:(ci + 1) * _CHUNK, :]
        cum_exc = incs[ci] - ohc + exc_chunk[ci:ci + 1, :]
        slot_f.append(jnp.sum(ohc * (cum_exc + base_lane), axis=1,
                              keepdims=True))
    slot = jnp.concatenate(slot_f, axis=0)
    slot_ref[...] = slot.astype(jnp.int32)

    bi = lax.broadcasted_iota(jnp.int32, (_NBMAX, _E), 0).astype(
        jnp.float32)
    bi_cl = jnp.minimum(bi, used[0, 0] - 1.0)
    be = jnp.sum(jnp.where(blk_start <= bi_cl, 1.0, 0.0), axis=1,
                 keepdims=True) - 1.0
    xi = bi_cl[:, 0:1]
    meta_ref[0:1, :] = used.astype(jnp.int32)
    meta_ref[1:1 + _NBMAX, :] = be.astype(jnp.int32)
    meta_ref[1 + _NBMAX:1 + 2 * _NBMAX, :] = xi.astype(jnp.int32)


def _swiglu(xb, wg, wu, wd):
    g = lax.dot_general(xb, wg, (((1,), (1,)), ((), ())),
                        preferred_element_type=jnp.float32)
    u = lax.dot_general(xb, wu, (((1,), (1,)), ((), ())),
                        preferred_element_type=jnp.float32)
    h = ((g * lax.logistic(g)) * u).astype(jnp.bfloat16)
    return lax.dot_general(h, wd, (((1,), (1,)), ((), ())),
                           preferred_element_type=jnp.float32)


def _grouped_ffn_kernel(meta_ref, x_ref, wg_ref, wu_ref, wd_ref, out_ref):
    i = pl.program_id(0)

    @pl.when(i < meta_ref[0])
    def _():
        out_ref[...] = _swiglu(x_ref[...].astype(jnp.bfloat16),
                               wg_ref[0].astype(jnp.bfloat16),
                               wu_ref[0].astype(jnp.bfloat16),
                               wd_ref[0].astype(jnp.bfloat16))


def _shared_kernel(x_ref, sg_ref, su_ref, sd_ref, out_ref):
    xb = x_ref[...].astype(jnp.bfloat16)
    out_ref[...] = _swiglu(xb, sg_ref[0].astype(jnp.bfloat16),
                           su_ref[0].astype(jnp.bfloat16),
                           sd_ref[0].astype(jnp.bfloat16))


def _combine_kernel(pv_ref1, pv_ref2, y1_ref, y2_ref, sh_ref, nw_ref,
                    out_ref):
    y = (pv_ref1[...] * y1_ref[...] + pv_ref2[...] * y2_ref[...]
         + sh_ref[...])
    rms = lax.rsqrt(jnp.mean(y * y, axis=-1, keepdims=True) + _EPS)
    out_ref[...] = y * rms * nw_ref[...]


_ROWS_PER_TILE = _P // 32
_SUB = 32
_NSUB = _ROWS_PER_TILE // _SUB


@functools.lru_cache(maxsize=None)
def _get_sc_kernels():
    mesh = plsc.VectorSubcoreMesh(core_axis_name="c", subcore_axis_name="s")

    @functools.partial(
        pl.kernel, mesh=mesh,
        out_type=jax.ShapeDtypeStruct((_CAP, _D), jnp.float32),
        scratch_types=[pltpu.VMEM((_NSUB, _SUB), jnp.int32),
                       pltpu.VMEM((_SUB, _D), jnp.float32),
                       pltpu.VMEM((_SUB, _D), jnp.float32),
                       pltpu.SemaphoreType.DMA,
                       pltpu.SemaphoreType.DMA,
                       pltpu.SemaphoreType.DMA,
                       pltpu.SemaphoreType.DMA],
    )
    def dispatch_sc(flat_hbm, slot_hbm, xs_hbm,
                    idx_v, rows0, rows1, sl0, sl1, ss0, ss1):
        wid = lax.axis_index("s") * 2 + lax.axis_index("c")
        tok0 = (wid % 16) * _ROWS_PER_TILE
        rows = (rows0, rows1)
        lsem = (sl0, sl1)
        ssem = (ss0, ss1)
        pltpu.sync_copy(slot_hbm.at[wid], idx_v)
        loads = [None, None]
        scat = [None, None]
        loads[0] = pltpu.async_copy(
            flat_hbm.at[pl.ds(tok0, _SUB)], rows0, lsem[0])
        for cc in range(_NSUB):
            b = cc % 2
            nb_ = (cc + 1) % 2
            loads[b].wait()
            if cc + 1 < _NSUB:
                if scat[nb_] is not None:
                    scat[nb_].wait()
                loads[nb_] = pltpu.async_copy(
                    flat_hbm.at[pl.ds(tok0 + (cc + 1) * _SUB, _SUB)],
                    rows[nb_], lsem[nb_])
            scat[b] = pltpu.async_copy(rows[b], xs_hbm.at[idx_v.at[cc]],
                                       ssem[b])
        for b in range(2):
            if scat[b] is not None:
                scat[b].wait()

    @functools.partial(
        pl.kernel, mesh=mesh,
        out_type=jax.ShapeDtypeStruct((_P, _D), jnp.float32),
        scratch_types=[pltpu.VMEM((_NSUB, _SUB), jnp.int32),
                       pltpu.VMEM((_SUB, _D), jnp.float32),
                       pltpu.VMEM((_SUB, _D), jnp.float32),
                       pltpu.SemaphoreType.DMA,
                       pltpu.SemaphoreType.DMA,
                       pltpu.SemaphoreType.DMA,
                       pltpu.SemaphoreType.DMA],
    )
    def combine_gather_sc(eo_hbm, slot_hbm, y12_hbm,
                          idx_v, rows0, rows1, sg0, sg1, sw0, sw1):
        wid = lax.axis_index("s") * 2 + lax.axis_index("c")
        base = wid * _ROWS_PER_TILE
        rows = (rows0, rows1)
        gsem = (sg0, sg1)
        wsem = (sw0, sw1)
        pltpu.sync_copy(slot_hbm.at[wid], idx_v)
        gath = [None, None]
        wr = [None, None]
        gath[0] = pltpu.async_copy(eo_hbm.at[idx_v.at[0]], rows0, gsem[0])
        for cc in range(_NSUB):
            b = cc % 2
            nb_ = (cc + 1) % 2
            gath[b].wait()
            if cc + 1 < _NSUB:
                if wr[nb_] is not None:
                    wr[nb_].wait()
                gath[nb_] = pltpu.async_copy(
                    eo_hbm.at[idx_v.at[cc + 1]], rows[nb_], gsem[nb_])
            wr[b] = pltpu.async_copy(
                rows[b], y12_hbm.at[pl.ds(base + cc * _SUB, _SUB)], wsem[b])
        for b in range(2):
            if wr[b] is not None:
                wr[b].wait()

    return dispatch_sc, combine_gather_sc


@jax.jit
def kernel(x, U, Wr, spec_bias, gen_bias, Wg, Wu, Wd, Sg, Su, Sd, norm_w):
    flat = x.reshape(_T, _D)
    u_col = U.reshape(_T, 1)

    pv, slot, meta, bal = pl.pallas_call(
        _router_kernel,
        out_shape=(
            jax.ShapeDtypeStruct((_P, 1), jnp.float32),
            jax.ShapeDtypeStruct((_P, 1), jnp.int32),
            jax.ShapeDtypeStruct((1 + 2 * _NBMAX, 1), jnp.int32),
            jax.ShapeDtypeStruct((1, 1), jnp.float32),
        ),
    )(flat, u_col, Wr, spec_bias.reshape(1, _E), gen_bias.reshape(1, _E))

    slot_tiled = slot.reshape(32, _NSUB, _SUB)
    meta_flat = meta.reshape(1 + 2 * _NBMAX)
    dispatch_sc, combine_gather_sc = _get_sc_kernels()

    x_sorted = dispatch_sc(flat, slot_tiled)

    shared_out = pl.pallas_call(
        _shared_kernel,
        grid=(2,),
        in_specs=[
            pl.BlockSpec((_T // 2, _D), lambda t: (t, 0)),
            pl.BlockSpec((1, _F, _D), lambda t: (0, 0, 0)),
            pl.BlockSpec((1, _F, _D), lambda t: (0, 0, 0)),
            pl.BlockSpec((1, _D, _F), lambda t: (0, 0, 0)),
        ],
        out_specs=pl.BlockSpec((_T // 2, _D), lambda t: (t, 0)),
        out_shape=jax.ShapeDtypeStruct((_T, _D), jnp.float32),
    )(flat, Sg, Su, Sd)

    nb1 = _NBMAX + 1
    eo_sorted = pl.pallas_call(
        _grouped_ffn_kernel,
        grid_spec=pltpu.PrefetchScalarGridSpec(
            num_scalar_prefetch=1,
            grid=(_NBMAX,),
            in_specs=[
                pl.BlockSpec((_BLK, _D), lambda i, m: (m[nb1 + i], 0)),
                pl.BlockSpec((1, _F, _D), lambda i, m: (m[1 + i], 0, 0)),
                pl.BlockSpec((1, _F, _D), lambda i, m: (m[1 + i], 0, 0)),
                pl.BlockSpec((1, _D, _F), lambda i, m: (m[1 + i], 0, 0)),
            ],
            out_specs=pl.BlockSpec((_BLK, _D), lambda i, m: (m[nb1 + i], 0)),
        ),
        out_shape=jax.ShapeDtypeStruct((_CAP, _D), jnp.float32),
    )(meta_flat, x_sorted, Wg, Wu, Wd)

    y12 = combine_gather_sc(eo_sorted, slot_tiled)

    y = pl.pallas_call(
        _combine_kernel,
        grid=(2,),
        in_specs=[
            pl.BlockSpec((_T // 2, 1), lambda t: (t, 0)),
            pl.BlockSpec((_T // 2, 1), lambda t: (t + 2, 0)),
            pl.BlockSpec((_T // 2, _D), lambda t: (t, 0)),
            pl.BlockSpec((_T // 2, _D), lambda t: (t + 2, 0)),
            pl.BlockSpec((_T // 2, _D), lambda t: (t, 0)),
            pl.BlockSpec((1, _D), lambda t: (0, 0)),
        ],
        out_specs=pl.BlockSpec((_T // 2, _D), lambda t: (t, 0)),
        out_shape=jax.ShapeDtypeStruct((_T, _D), jnp.float32),
    )(pv, pv, y12, y12, shared_out, norm_w.reshape(1, _D))

    return (y.reshape(x.shape), bal.reshape(()))


# speedup vs baseline: 1.3381x; 1.0011x over previous
import functools

import jax
import jax.numpy as jnp
from jax import lax
from jax.experimental import pallas as pl
from jax.experimental.pallas import tpu as pltpu
from jax.experimental.pallas import tpu_sc as plsc

_T = 2048
_D = 1024
_F = 1024
_E = 8
_P = 2 * _T
_BLK = 512
_NBMAX = (_P // _BLK) + _E - 1
_CAP = _NBMAX * _BLK
_CHUNK = 512
_NCHUNK = _P // _CHUNK
_LOAD_COEF = 0.01
_EPS = 1e-6


def _router_kernel(flat_ref, u_ref, wr_ref, sb_ref, gb_ref,
                   pv_ref, slot_ref, meta_ref, bal_ref):
    flat = flat_ref[...]
    logits = lax.dot_general(flat, wr_ref[...], (((1,), (1,)), ((), ())),
                             preferred_element_type=jnp.float32)
    u = jnp.clip(u_ref[...], 0.0, 1.0)
    logits = logits + u * sb_ref[...] + (1.0 - u) * gb_ref[...]
    m = jnp.max(logits, axis=-1, keepdims=True)
    ex = jnp.exp(logits - m)
    probs = ex / jnp.sum(ex, axis=-1, keepdims=True)

    lane = lax.broadcasted_iota(jnp.int32, probs.shape, 1)
    m1 = jnp.max(probs, axis=-1, keepdims=True)
    i1 = jnp.min(jnp.where(probs == m1, lane, _E), axis=-1, keepdims=True)
    oh1 = (lane == i1)
    p_rest = jnp.where(oh1, -jnp.inf, probs)
    m2 = jnp.max(p_rest, axis=-1, keepdims=True)
    i2 = jnp.min(jnp.where(p_rest == m2, lane, _E), axis=-1, keepdims=True)
    oh2 = (lane == i2)

    pv_ref[0:_T, :] = m1
    pv_ref[_T:_P, :] = m2

    colmean = jnp.mean(probs, axis=0, keepdims=True)
    bal_ref[...] = _E * _LOAD_COEF * jnp.sum(colmean * colmean, keepdims=True)

    oh = jnp.concatenate([jnp.where(oh1, 1.0, 0.0),
                          jnp.where(oh2, 1.0, 0.0)], axis=0)

    r = lax.broadcasted_iota(jnp.int32, (_CHUNK, _CHUNK), 0)
    c = lax.broadcasted_iota(jnp.int32, (_CHUNK, _CHUNK), 1)
    ltri = jnp.where(r >= c, 1.0, 0.0)

    incs = []
    tails = []
    for ci in range(_NCHUNK):
        chunk = oh[ci * _CHUNK:(ci + 1) * _CHUNK, :]
        inc = jnp.dot(ltri, chunk, preferred_element_type=jnp.float32)
        incs.append(inc)
        tails.append(inc[_CHUNK - 1:_CHUNK, :])
    totals = jnp.concatenate(tails, axis=0)

    r2 = lax.broadcasted_iota(jnp.int32, (_NCHUNK, _NCHUNK), 0)
    c2 = lax.broadcasted_iota(jnp.int32, (_NCHUNK, _NCHUNK), 1)
    sltri = jnp.where(r2 > c2, 1.0, 0.0)
    exc_chunk = jnp.dot(sltri, totals, preferred_element_type=jnp.float32)

    counts = jnp.sum(totals, axis=0, keepdims=True)
    nb = jnp.floor((counts + float(_BLK - 1)) * (1.0 / _BLK))
    eye_c = lax.broadcasted_iota(jnp.int32, (_E, _E), 0)
    eye_r = lax.broadcasted_iota(jnp.int32, (_E, _E), 1)
    supper = jnp.where(eye_c < eye_r, 1.0, 0.0)
    blk_start = jnp.dot(nb, supper, preferred_element_type=jnp.float32)
    used = jnp.sum(nb, axis=1, keepdims=True)

    base_lane = blk_start * float(_BLK)
    slot_f = []
    for ci in range(_NCHUNK):
        ohc = oh[ci * _CHUNK:(ci + 1) * _CHUNK, :]
        cum_exc = incs[ci] - ohc + exc_chunk[ci:ci + 1, :]
        slot_f.append(jnp.sum(ohc * (cum_exc + base_lane), axis=1,
                              keepdims=True))
    slot = jnp.concatenate(slot_f, axis=0)
    slot_ref[...] = slot.astype(jnp.int32)

    bi = lax.broadcasted_iota(jnp.int32, (_NBMAX, _E), 0).astype(
        jnp.float32)
    bi_cl = jnp.minimum(bi, used[0, 0] - 1.0)
    be = jnp.sum(jnp.where(blk_start <= bi_cl, 1.0, 0.0), axis=1,
                 keepdims=True) - 1.0
    xi = bi_cl[:, 0:1]
    meta_ref[0:1, :] = used.astype(jnp.int32)
    meta_ref[1:1 + _NBMAX, :] = be.astype(jnp.int32)
    meta_ref[1 + _NBMAX:1 + 2 * _NBMAX, :] = xi.astype(jnp.int32)


def _swiglu(xb, wg, wu, wd):
    g = lax.dot_general(xb, wg, (((1,), (1,)), ((), ())),
                        preferred_element_type=jnp.float32)
    u = lax.dot_general(xb, wu, (((1,), (1,)), ((), ())),
                        preferred_element_type=jnp.float32)
    h = ((g * lax.logistic(g)) * u).astype(jnp.bfloat16)
    return lax.dot_general(h, wd, (((1,), (1,)), ((), ())),
                           preferred_element_type=jnp.float32)


def _grouped_ffn_kernel(meta_ref, x_ref, wg_ref, wu_ref, wd_ref, out_ref):
    i = pl.program_id(0)

    @pl.when(i < meta_ref[0])
    def _():
        out_ref[...] = _swiglu(x_ref[...].astype(jnp.bfloat16),
                               wg_ref[0].astype(jnp.bfloat16),
                               wu_ref[0].astype(jnp.bfloat16),
                               wd_ref[0].astype(jnp.bfloat16))


def _shared_kernel(x_ref, sg_ref, su_ref, sd_ref, out_ref):
    xb = x_ref[...].astype(jnp.bfloat16)
    out_ref[...] = _swiglu(xb, sg_ref[0].astype(jnp.bfloat16),
                           su_ref[0].astype(jnp.bfloat16),
                           sd_ref[0].astype(jnp.bfloat16))


def _combine_kernel(pv_ref1, pv_ref2, y1_ref, y2_ref, sh_ref, nw_ref,
                    out_ref):
    y = (pv_ref1[...] * y1_ref[...] + pv_ref2[...] * y2_ref[...]
         + sh_ref[...])
    rms = lax.rsqrt(jnp.mean(y * y, axis=-1, keepdims=True) + _EPS)
    out_ref[...] = y * rms * nw_ref[...]


_ROWS_PER_TILE = _P // 32
_SUB = 32
_NSUB = _ROWS_PER_TILE // _SUB


@functools.lru_cache(maxsize=None)
def _get_sc_kernels():
    mesh = plsc.VectorSubcoreMesh(core_axis_name="c", subcore_axis_name="s")

    @functools.partial(
        pl.kernel, mesh=mesh,
        out_type=jax.ShapeDtypeStruct((_CAP, _D), jnp.float32),
        scratch_types=[pltpu.VMEM((_NSUB, _SUB), jnp.int32),
                       pltpu.VMEM((_SUB, _D), jnp.float32),
                       pltpu.VMEM((_SUB, _D), jnp.float32),
                       pltpu.SemaphoreType.DMA,
                       pltpu.SemaphoreType.DMA,
                       pltpu.SemaphoreType.DMA,
                       pltpu.SemaphoreType.DMA],
    )
    def dispatch_sc(flat_hbm, slot_hbm, xs_hbm,
                    idx_v, rows0, rows1, sl0, sl1, ss0, ss1):
        wid = lax.axis_index("s") * 2 + lax.axis_index("c")
        tok0 = (wid % 16) * _ROWS_PER_TILE
        rows = (rows0, rows1)
        lsem = (sl0, sl1)
        ssem = (ss0, ss1)
        pltpu.sync_copy(slot_hbm.at[wid], idx_v)
        loads = [None, None]
        scat = [None, None]
        loads[0] = pltpu.async_copy(
            flat_hbm.at[pl.ds(tok0, _SUB)], rows0, lsem[0])
        for cc in range(_NSUB):
            b = cc % 2
            nb_ = (cc + 1) % 2
            loads[b].wait()
            if cc + 1 < _NSUB:
                if scat[nb_] is not None:
                    scat[nb_].wait()
                loads[nb_] = pltpu.async_copy(
                    flat_hbm.at[pl.ds(tok0 + (cc + 1) * _SUB, _SUB)],
                    rows[nb_], lsem[nb_])
            scat[b] = pltpu.async_copy(rows[b], xs_hbm.at[idx_v.at[cc]],
                                       ssem[b])
        for b in range(2):
            if scat[b] is not None:
                scat[b].wait()

    @functools.partial(
        pl.kernel, mesh=mesh,
        out_type=jax.ShapeDtypeStruct((_P, _D), jnp.float32),
        scratch_types=[pltpu.VMEM((_NSUB, _SUB), jnp.int32),
                       pltpu.VMEM((_SUB, _D), jnp.float32),
                       pltpu.VMEM((_SUB, _D), jnp.float32),
                       pltpu.SemaphoreType.DMA,
                       pltpu.SemaphoreType.DMA,
                       pltpu.SemaphoreType.DMA,
                       pltpu.SemaphoreType.DMA],
    )
    def combine_gather_sc(eo_hbm, slot_hbm, y12_hbm,
                          idx_v, rows0, rows1, sg0, sg1, sw0, sw1):
        wid = lax.axis_index("s") * 2 + lax.axis_index("c")
        base = wid * _ROWS_PER_TILE
        rows = (rows0, rows1)
        gsem = (sg0, sg1)
        wsem = (sw0, sw1)
        pltpu.sync_copy(slot_hbm.at[wid], idx_v)
        gath = [None, None]
        wr = [None, None]
        gath[0] = pltpu.async_copy(eo_hbm.at[idx_v.at[0]], rows0, gsem[0])
        for cc in range(_NSUB):
            b = cc % 2
            nb_ = (cc + 1) % 2
            gath[b].wait()
            if cc + 1 < _NSUB:
                if wr[nb_] is not None:
                    wr[nb_].wait()
                gath[nb_] = pltpu.async_copy(
                    eo_hbm.at[idx_v.at[cc + 1]], rows[nb_], gsem[nb_])
            wr[b] = pltpu.async_copy(
                rows[b], y12_hbm.at[pl.ds(base + cc * _SUB, _SUB)], wsem[b])
        for b in range(2):
            if wr[b] is not None:
                wr[b].wait()

    return dispatch_sc, combine_gather_sc


@jax.jit
def kernel(x, U, Wr, spec_bias, gen_bias, Wg, Wu, Wd, Sg, Su, Sd, norm_w):
    flat = x.reshape(_T, _D)
    u_col = U.reshape(_T, 1)

    pv, slot, meta, bal = pl.pallas_call(
        _router_kernel,
        out_shape=(
            jax.ShapeDtypeStruct((_P, 1), jnp.float32),
            jax.ShapeDtypeStruct((_P, 1), jnp.int32),
            jax.ShapeDtypeStruct((1 + 2 * _NBMAX, 1), jnp.int32),
            jax.ShapeDtypeStruct((1, 1), jnp.float32),
        ),
    )(flat, u_col, Wr, spec_bias.reshape(1, _E), gen_bias.reshape(1, _E))

    slot_tiled = slot.reshape(32, _NSUB, _SUB)
    meta_flat = meta.reshape(1 + 2 * _NBMAX)
    dispatch_sc, combine_gather_sc = _get_sc_kernels()

    x_sorted = dispatch_sc(flat, slot_tiled)

    shared_out = pl.pallas_call(
        _shared_kernel,
        grid=(2,),
        in_specs=[
            pl.BlockSpec((_T // 2, _D), lambda t: (t, 0)),
            pl.BlockSpec((1, _F, _D), lambda t: (0, 0, 0)),
            pl.BlockSpec((1, _F, _D), lambda t: (0, 0, 0)),
            pl.BlockSpec((1, _D, _F), lambda t: (0, 0, 0)),
        ],
        out_specs=pl.BlockSpec((_T // 2, _D), lambda t: (t, 0)),
        out_shape=jax.ShapeDtypeStruct((_T, _D), jnp.float32),
    )(flat, Sg, Su, Sd)

    nb1 = _NBMAX + 1
    eo_sorted = pl.pallas_call(
        _grouped_ffn_kernel,
        grid_spec=pltpu.PrefetchScalarGridSpec(
            num_scalar_prefetch=1,
            grid=(_NBMAX,),
            in_specs=[
                pl.BlockSpec((_BLK, _D), lambda i, m: (m[nb1 + i], 0)),
                pl.BlockSpec((1, _F, _D), lambda i, m: (m[1 + i], 0, 0)),
                pl.BlockSpec((1, _F, _D), lambda i, m: (m[1 + i], 0, 0)),
                pl.BlockSpec((1, _D, _F), lambda i, m: (m[1 + i], 0, 0)),
            ],
            out_specs=pl.BlockSpec((_BLK, _D), lambda i, m: (m[nb1 + i], 0)),
        ),
        out_shape=jax.ShapeDtypeStruct((_CAP, _D), jnp.float32),
    )(meta_flat, x_sorted, Wg, Wu, Wd)

    y12 = combine_gather_sc(eo_sorted, slot_tiled)

    y = pl.pallas_call(
        _combine_kernel,
        grid=(2,),
        in_specs=[
            pl.BlockSpec((_T // 2, 1), lambda t: (t, 0)),
            pl.BlockSpec((_T // 2, 1), lambda t: (t + 2, 0)),
            pl.BlockSpec((_T // 2, _D), lambda t: (t, 0)),
            pl.BlockSpec((_T // 2, _D), lambda t: (t + 2, 0)),
            pl.BlockSpec((_T // 2, _D), lambda t: (t, 0)),
            pl.BlockSpec((1, _D), lambda t: (0, 0)),
        ],
        out_specs=pl.BlockSpec((_T // 2, _D), lambda t: (t, 0)),
        out_shape=jax.ShapeDtypeStruct((_T, _D), jnp.float32),
    )(pv, pv, y12, y12, shared_out, norm_w.reshape(1, _D))

    return (y.reshape(x.shape), bal.reshape(()))
